# speedup vs baseline: 1.0001x; 1.0001x over previous
import jax
import jax.numpy as jnp
from jax.experimental import pallas as pl

N_NODES = 10000
N_EDGES = 320000
L = 4


def _bn(x, gamma, beta, eps=1e-5):
    mu = jnp.mean(x, axis=0)
    var = jnp.var(x, axis=0)
    return (x - mu) / jnp.sqrt(var + eps) * gamma + beta


def kernel(nodes_feat, edges_feat, nodes_num_norm_sqrt, edges_num_norm_sqrt, edge_index, Wh, bh, We, be, AW, Ab, BW, Bb, CW, Cb, DW, Db, EW, Eb, bn_h_gamma, bn_h_beta, bn_e_gamma, bn_e_beta, W0, b0, W1, b1, W2, b2):
    src = edge_index[0]
    dst = edge_index[1]
    h = nodes_feat @ Wh + bh
    e = edges_feat @ We + be
    for l in range(L):
        h_in = h
        e_in = e
        Ah = h @ AW[l] + Ab[l]
        Bh = h @ BW[l] + Bb[l]
        Dh = h @ DW[l] + Db[l]
        Eh = h @ EW[l] + Eb[l]
        Ce = e @ CW[l] + Cb[l]
        e_new = Ce + Dh[src] + Eh[dst]
        sigma = jax.nn.sigmoid(e_new)
        num = jax.ops.segment_sum(sigma * Bh[src], dst, num_segments=N_NODES)
        den = jax.ops.segment_sum(sigma, dst, num_segments=N_NODES)
        h_new = Ah + num / (den + 1e-6)
        h_new = h_new * nodes_num_norm_sqrt
        e_new = e_new * edges_num_norm_sqrt
        h_new = _bn(h_new, bn_h_gamma[l], bn_h_beta[l])
        e_new = _bn(e_new, bn_e_gamma[l], bn_e_beta[l])
        h_new = jax.nn.relu(h_new)
        e_new = jax.nn.relu(e_new)
        h = h_in + h_new
        e = e_in + e_new
    hg = jnp.mean(h, axis=0)
    y = jax.nn.relu(hg @ W0 + b0)
    y = jax.nn.relu(y @ W1 + b1)
    logits = y @ W2 + b2
    return logits


# speedup vs baseline: 3.0974x; 3.0971x over previous
import functools

import jax
import jax.numpy as jnp
from jax import lax
from jax.experimental import pallas as pl
from jax.experimental.pallas import tpu as pltpu
from jax.experimental.pallas import tpu_sc as plsc

N = 10000
E = 320000
W = 128
WD = 256
L = 4
BE = 4000
NBLK = E // BE
NW = 32
CH = 1280
NCHUNK = E // CH
CPW = -(-NCHUNK // NW)
CPS = -(-NCHUNK // 16)
NST = CH // W


def _sc_gather(gtab, etab, src3, dst3):
    mesh = plsc.VectorSubcoreMesh(core_axis_name="c", subcore_axis_name="s")

    @functools.partial(
        pl.kernel,
        out_type=(jax.ShapeDtypeStruct((E, WD), jnp.float32),
                  jax.ShapeDtypeStruct((E, W), jnp.float32)),
        mesh=mesh,
        scratch_types=[
            pltpu.VMEM((NST, 128), jnp.int32),
            pltpu.VMEM((NST, 128), jnp.int32),
            pltpu.VMEM((128, WD), jnp.float32),
            pltpu.VMEM((128, W), jnp.float32),
            pltpu.SemaphoreType.DMA,
            pltpu.SemaphoreType.DMA,
        ],
    )
    def k(gtab_hbm, etab_hbm, src_hbm, dst_hbm, gout_hbm, eout_hbm,
          sidx, didx, grows, erows, sem1, sem2):
        wid = lax.axis_index("c") * 16 + lax.axis_index("s")

        @pl.loop(0, CPW)
        def _(i):
            c = wid + i * NW

            @pl.when(c < NCHUNK)
            def _():
                pltpu.sync_copy(src_hbm.at[c], sidx)
                pltpu.sync_copy(dst_hbm.at[c], didx)
                for j in range(NST):
                    cp1 = pltpu.async_copy(
                        gtab_hbm.at[sidx.at[j]], grows, sem1)
                    cp2 = pltpu.async_copy(
                        etab_hbm.at[didx.at[j]], erows, sem2)
                    cp1.wait()
                    cp2.wait()
                    base = c * CH + j * 128
                    pltpu.sync_copy(grows, gout_hbm.at[pl.ds(base, 128)])
                    pltpu.sync_copy(erows, eout_hbm.at[pl.ds(base, 128)])

    return k(gtab, etab, src3, dst3)


def _sc_scatter(m, dst3, zer):
    mesh = plsc.VectorSubcoreMesh(core_axis_name="c", subcore_axis_name="s")

    @functools.partial(
        pl.kernel,
        out_type=jax.ShapeDtypeStruct((2, N, W), jnp.float32),
        mesh=mesh,
        scratch_types=[
            pltpu.VMEM((NST, 128), jnp.int32),
            pltpu.VMEM((128, W), jnp.float32),
            pltpu.VMEM_SHARED((N, W), jnp.float32),
            pltpu.SemaphoreType.DMA,
        ],
    )
    def k(m_hbm, dst_hbm, z_hbm, p_hbm, didx, mrows, accum, sem):
        core = lax.axis_index("c")
        s = lax.axis_index("s")

        @pl.when(s == 0)
        def _():
            pltpu.sync_copy(z_hbm, accum)

        plsc.subcore_barrier()

        @pl.loop(0, CPS)
        def _(i):
            c = s + i * 16

            @pl.when(c < NCHUNK)
            def _():
                pltpu.sync_copy(dst_hbm.at[c], didx)
                for j in range(NST):
                    pltpu.sync_copy(
                        m_hbm.at[core, pl.ds(c * CH + j * 128, 128)], mrows)
                    pltpu.sync_copy(mrows, accum.at[didx.at[j]], add=True)

        plsc.subcore_barrier()

        @pl.when(s < 10)
        def _():
            pltpu.sync_copy(accum.at[pl.ds(s * 1000, 1000)],
                            p_hbm.at[core, pl.ds(s * 1000, 1000)])

    return k(m, dst3, zer)


def _embed_h(nodes_feat, whp, bhp):
    def body(x_ref, w_ref, b_ref, o_ref):
        o_ref[...] = jnp.dot(x_ref[...], w_ref[...],
                             preferred_element_type=jnp.float32) + b_ref[...]
    return pl.pallas_call(
        body, out_shape=jax.ShapeDtypeStruct((N, W), jnp.float32),
    )(nodes_feat, whp, bhp)


def _embed_e(ef, wep, bep):
    def body(f_ref, w_ref, b_ref, o_ref):
        o_ref[...] = f_ref[...] * w_ref[...] + b_ref[...]
    return pl.pallas_call(
        body,
        grid=(NBLK,),
        in_specs=[pl.BlockSpec((BE, 1), lambda i: (i, 0)),
                  pl.BlockSpec((1, W), lambda i: (0, 0)),
                  pl.BlockSpec((1, W), lambda i: (0, 0))],
        out_specs=pl.BlockSpec((BE, W), lambda i: (i, 0)),
        out_shape=jax.ShapeDtypeStruct((E, W), jnp.float32),
    )(ef, wep, bep)


def _node_mm(h, wc, bc):
    def body(h_ref, w_ref, b_ref, ah_ref, g_ref, eh_ref):
        hw = jnp.dot(h_ref[...], w_ref[...],
                     preferred_element_type=jnp.float32) + b_ref[...]
        ah_ref[...] = hw[:, 0:W]
        g_ref[:, 0:W] = hw[:, 2 * W:3 * W]
        g_ref[:, W:2 * W] = hw[:, W:2 * W]
        eh_ref[...] = hw[:, 3 * W:4 * W]
    return pl.pallas_call(
        body,
        out_shape=(jax.ShapeDtypeStruct((N, W), jnp.float32),
                   jax.ShapeDtypeStruct((N, WD), jnp.float32),
                   jax.ShapeDtypeStruct((N, W), jnp.float32)),
    )(h, wc, bc)


def _edge_main(e, gsrc, edst, enorm, cw, cb):
    def body(e_ref, g_ref, ed_ref, n_ref, w_ref, b_ref, y_ref, m_ref, st_ref):
        i = pl.program_id(0)
        ce = jnp.dot(e_ref[...], w_ref[...],
                     preferred_element_type=jnp.float32) + b_ref[...]
        x = ce + g_ref[:, 0:W] + ed_ref[...]
        sig = jax.nn.sigmoid(x)
        y = x * n_ref[...]
        y_ref[...] = y
        m_ref[0] = sig
        m_ref[1] = sig * g_ref[:, W:2 * W]
        s1 = jnp.sum(y, axis=0, keepdims=True)
        s2 = jnp.sum(y * y, axis=0, keepdims=True)
        part = jnp.concatenate([s1, s2, jnp.zeros((6, W), jnp.float32)], axis=0)

        @pl.when(i == 0)
        def _():
            st_ref[...] = part

        @pl.when(i > 0)
        def _():
            st_ref[...] += part

    return pl.pallas_call(
        body,
        grid=(NBLK,),
        in_specs=[pl.BlockSpec((BE, W), lambda i: (i, 0)),
                  pl.BlockSpec((BE, WD), lambda i: (i, 0)),
                  pl.BlockSpec((BE, W), lambda i: (i, 0)),
                  pl.BlockSpec((BE, 1), lambda i: (i, 0)),
                  pl.BlockSpec((W, W), lambda i: (0, 0)),
                  pl.BlockSpec((1, W), lambda i: (0, 0))],
        out_specs=(pl.BlockSpec((BE, W), lambda i: (i, 0)),
                   pl.BlockSpec((2, BE, W), lambda i: (0, i, 0)),
                   pl.BlockSpec((8, W), lambda i: (0, 0))),
        out_shape=(jax.ShapeDtypeStruct((E, W), jnp.float32),
                   jax.ShapeDtypeStruct((2, E, W), jnp.float32),
                   jax.ShapeDtypeStruct((8, W), jnp.float32)),
    )(e, gsrc, edst, enorm, cw, cb)


def _node_finish(ah, p, nnorm, h_in, gam, bet):
    def body(ah_ref, p_ref, nn_ref, h_ref, g_ref, b_ref, o_ref):
        den = p_ref[0]
        num = p_ref[1]
        hn = (ah_ref[...] + num / (den + 1e-6)) * nn_ref[...]
        mu = jnp.mean(hn, axis=0, keepdims=True)
        var = jnp.mean(hn * hn, axis=0, keepdims=True) - mu * mu
        bn = (hn - mu) / jnp.sqrt(var + 1e-5) * g_ref[...] + b_ref[...]
        o_ref[...] = h_ref[...] + jnp.maximum(bn, 0.0)
    return pl.pallas_call(
        body, out_shape=jax.ShapeDtypeStruct((N, W), jnp.float32),
    )(ah, p, nnorm, h_in, gam, bet)


def _edge_finish(y, e_in, st, gam, bet):
    def body(y_ref, e_ref, st_ref, g_ref, b_ref, o_ref):
        mu = st_ref[0:1, :] * (1.0 / E)
        var = st_ref[1:2, :] * (1.0 / E) - mu * mu
        bn = (y_ref[...] - mu) / jnp.sqrt(var + 1e-5) * g_ref[...] + b_ref[...]
        o_ref[...] = e_ref[...] + jnp.maximum(bn, 0.0)
    return pl.pallas_call(
        body,
        grid=(NBLK,),
        in_specs=[pl.BlockSpec((BE, W), lambda i: (i, 0)),
                  pl.BlockSpec((BE, W), lambda i: (i, 0)),
                  pl.BlockSpec((8, W), lambda i: (0, 0)),
                  pl.BlockSpec((1, W), lambda i: (0, 0)),
                  pl.BlockSpec((1, W), lambda i: (0, 0))],
        out_specs=pl.BlockSpec((BE, W), lambda i: (i, 0)),
        out_shape=jax.ShapeDtypeStruct((E, W), jnp.float32),
    )(y, e_in, st, gam, bet)


def _readout(h, w0, b0_, w1, b1_, w2, b2_):
    def body(h_ref, w0_ref, b0_ref, w1_ref, b1_ref, w2_ref, b2_ref, o_ref):
        hg = jnp.mean(h_ref[...], axis=0, keepdims=True)
        y0 = jnp.maximum(jnp.dot(hg, w0_ref[...],
                                 preferred_element_type=jnp.float32) + b0_ref[...], 0.0)
        y1 = jnp.maximum(jnp.dot(y0, w1_ref[...],
                                 preferred_element_type=jnp.float32) + b1_ref[...], 0.0)
        o_ref[...] = jnp.dot(y1, w2_ref[...],
                             preferred_element_type=jnp.float32) + b2_ref[...]
    return pl.pallas_call(
        body, out_shape=jax.ShapeDtypeStruct((1, 128), jnp.float32),
    )(h, w0, b0_, w1, b1_, w2, b2_)


def _padw(w, r, c):
    return jnp.zeros((r, c), jnp.float32).at[:w.shape[0], :w.shape[1]].set(w)


def _padb(b, c):
    return jnp.zeros((1, c), jnp.float32).at[0, :b.shape[0]].set(b)


def kernel(nodes_feat, edges_feat, nodes_num_norm_sqrt, edges_num_norm_sqrt, edge_index, Wh, bh, We, be, AW, Ab, BW, Bb, CW, Cb, DW, Db, EW, Eb, bn_h_gamma, bn_h_beta, bn_e_gamma, bn_e_beta, W0, b0, W1, b1, W2, b2):
    src3 = edge_index[0].reshape(NCHUNK, NST, 128)
    dst3 = edge_index[1].reshape(NCHUNK, NST, 128)

    h = _embed_h(nodes_feat, _padw(Wh, 128, W), _padb(bh, W))
    e = _embed_e(edges_feat, _padw(We, 1, W), _padb(be, W))
    zer = jnp.zeros((N, W), jnp.float32)

    for l in range(L):
        wc = jnp.concatenate([_padw(AW[l], W, W), _padw(BW[l], W, W),
                              _padw(DW[l], W, W), _padw(EW[l], W, W)], axis=1)
        bc = jnp.concatenate([_padb(Ab[l], W), _padb(Bb[l], W),
                              _padb(Db[l], W), _padb(Eb[l], W)], axis=1)
        ah, g, eh = _node_mm(h, wc, bc)
        gsrc, edst_ = _sc_gather(g, eh, src3, dst3)
        y, m, st = _edge_main(e, gsrc, edst_, edges_num_norm_sqrt,
                              _padw(CW[l], W, W), _padb(Cb[l], W))
        p = _sc_scatter(m, dst3, zer)
        h = _node_finish(ah, p, nodes_num_norm_sqrt, h,
                         _padb(bn_h_gamma[l], W), _padb(bn_h_beta[l], W))
        e = _edge_finish(y, e, st, _padb(bn_e_gamma[l], W), _padb(bn_e_beta[l], W))

    out = _readout(h, _padw(W0, W, 128), _padb(b0, 128),
                   _padw(W1, 128, 128), _padb(b1, 128),
                   _padw(W2, 128, 128), _padb(b2, 128))
    return out[0, :10]


# speedup vs baseline: 3.3757x; 1.0898x over previous
import functools

import jax
import jax.numpy as jnp
from jax import lax
from jax.experimental import pallas as pl
from jax.experimental.pallas import tpu as pltpu
from jax.experimental.pallas import tpu_sc as plsc

N = 10000
E = 320000
W = 128
WD = 256
L = 4
BE = 4000
NBLK = E // BE
NW = 32
CH = 1280
NCHUNK = E // CH
CPW = -(-NCHUNK // NW)
CPS = -(-NCHUNK // 16)
NST = CH // W


def _sc_gather(gtab, etab, src3, dst3):
    mesh = plsc.VectorSubcoreMesh(core_axis_name="c", subcore_axis_name="s")

    @functools.partial(
        pl.kernel,
        out_type=(jax.ShapeDtypeStruct((E, WD), jnp.float32),
                  jax.ShapeDtypeStruct((E, W), jnp.float32)),
        mesh=mesh,
        scratch_types=[
            pltpu.VMEM((NST, 128), jnp.int32),
            pltpu.VMEM((NST, 128), jnp.int32),
            pltpu.VMEM((2, 128, WD), jnp.float32),
            pltpu.VMEM((2, 128, W), jnp.float32),
            pltpu.SemaphoreType.DMA,
            pltpu.SemaphoreType.DMA,
            pltpu.SemaphoreType.DMA,
        ],
    )
    def k(gtab_hbm, etab_hbm, src_hbm, dst_hbm, gout_hbm, eout_hbm,
          sidx, didx, grows, erows, semg, seme, semw):
        wid = lax.axis_index("c") * 16 + lax.axis_index("s")

        @pl.loop(0, CPW)
        def _(i):
            c = wid + i * NW

            @pl.when(c < NCHUNK)
            def _():
                pltpu.sync_copy(src_hbm.at[c], sidx)
                pltpu.sync_copy(dst_hbm.at[c], didx)

                def gath(j):
                    b = j % 2
                    return (pltpu.async_copy(
                                gtab_hbm.at[sidx.at[j]], grows.at[b], semg),
                            pltpu.async_copy(
                                etab_hbm.at[didx.at[j]], erows.at[b], seme))

                def wr(j):
                    b = j % 2
                    base = c * CH + j * 128
                    return (pltpu.async_copy(
                                grows.at[b], gout_hbm.at[pl.ds(base, 128)], semw),
                            pltpu.async_copy(
                                erows.at[b], eout_hbm.at[pl.ds(base, 128)], semw))

                gcur = gath(0)
                wprev = None
                for j in range(NST):
                    for cp in gcur:
                        cp.wait()
                    if wprev is not None:
                        for cp in wprev:
                            cp.wait()
                    if j + 1 < NST:
                        gnext = gath(j + 1)
                    wcur = wr(j)
                    if j + 1 < NST:
                        gcur = gnext
                    wprev = wcur
                for cp in wprev:
                    cp.wait()

    return k(gtab, etab, src3, dst3)


def _sc_scatter(m, dst3, zer):
    mesh = plsc.VectorSubcoreMesh(core_axis_name="c", subcore_axis_name="s")

    @functools.partial(
        pl.kernel,
        out_type=jax.ShapeDtypeStruct((2, N, W), jnp.float32),
        mesh=mesh,
        scratch_types=[
            pltpu.VMEM((NST, 128), jnp.int32),
            pltpu.VMEM((2, 128, W), jnp.float32),
            pltpu.VMEM_SHARED((N, W), jnp.float32),
            pltpu.SemaphoreType.DMA,
            pltpu.SemaphoreType.DMA,
        ],
    )
    def k(m_hbm, dst_hbm, z_hbm, p_hbm, didx, mrows, accum, semld, semsc):
        core = lax.axis_index("c")
        s = lax.axis_index("s")

        @pl.when(s == 0)
        def _():
            pltpu.sync_copy(z_hbm, accum)

        plsc.subcore_barrier()

        @pl.loop(0, CPS)
        def _(i):
            c = s + i * 16

            @pl.when(c < NCHUNK)
            def _():
                pltpu.sync_copy(dst_hbm.at[c], didx)

                def mload(j):
                    return pltpu.async_copy(
                        m_hbm.at[core, pl.ds(c * CH + j * 128, 128)],
                        mrows.at[j % 2], semld)

                lcur = mload(0)
                scprev = None
                for j in range(NST):
                    lcur.wait()
                    if scprev is not None:
                        scprev.wait()
                    if j + 1 < NST:
                        lnext = mload(j + 1)
                    sccur = pltpu.async_copy(
                        mrows.at[j % 2], accum.at[didx.at[j]], semsc, add=True)
                    if j + 1 < NST:
                        lcur = lnext
                    scprev = sccur
                scprev.wait()

        plsc.subcore_barrier()

        @pl.when(s < 10)
        def _():
            pltpu.sync_copy(accum.at[pl.ds(s * 1000, 1000)],
                            p_hbm.at[core, pl.ds(s * 1000, 1000)])

    return k(m, dst3, zer)


def _embed_h(nodes_feat, whp, bhp):
    def body(x_ref, w_ref, b_ref, o_ref):
        o_ref[...] = jnp.dot(x_ref[...], w_ref[...],
                             preferred_element_type=jnp.float32) + b_ref[...]
    return pl.pallas_call(
        body, out_shape=jax.ShapeDtypeStruct((N, W), jnp.float32),
    )(nodes_feat, whp, bhp)


def _embed_e(ef, wep, bep):
    def body(f_ref, w_ref, b_ref, o_ref):
        o_ref[...] = f_ref[...] * w_ref[...] + b_ref[...]
    return pl.pallas_call(
        body,
        grid=(NBLK,),
        in_specs=[pl.BlockSpec((BE, 1), lambda i: (i, 0)),
                  pl.BlockSpec((1, W), lambda i: (0, 0)),
                  pl.BlockSpec((1, W), lambda i: (0, 0))],
        out_specs=pl.BlockSpec((BE, W), lambda i: (i, 0)),
        out_shape=jax.ShapeDtypeStruct((E, W), jnp.float32),
    )(ef, wep, bep)


def _node_mm(h, wc, bc):
    def body(h_ref, w_ref, b_ref, ah_ref, g_ref, eh_ref):
        hw = jnp.dot(h_ref[...], w_ref[...],
                     preferred_element_type=jnp.float32) + b_ref[...]
        ah_ref[...] = hw[:, 0:W]
        g_ref[:, 0:W] = hw[:, 2 * W:3 * W]
        g_ref[:, W:2 * W] = hw[:, W:2 * W]
        eh_ref[...] = hw[:, 3 * W:4 * W]
    return pl.pallas_call(
        body,
        out_shape=(jax.ShapeDtypeStruct((N, W), jnp.float32),
                   jax.ShapeDtypeStruct((N, WD), jnp.float32),
                   jax.ShapeDtypeStruct((N, W), jnp.float32)),
    )(h, wc, bc)


def _edge_main(e, gsrc, edst, enorm, cw, cb):
    def body(e_ref, g_ref, ed_ref, n_ref, w_ref, b_ref, y_ref, m_ref, st_ref):
        i = pl.program_id(0)
        ce = jnp.dot(e_ref[...], w_ref[...],
                     preferred_element_type=jnp.float32) + b_ref[...]
        x = ce + g_ref[:, 0:W] + ed_ref[...]
        sig = jax.nn.sigmoid(x)
        y = x * n_ref[...]
        y_ref[...] = y
        m_ref[0] = sig
        m_ref[1] = sig * g_ref[:, W:2 * W]
        s1 = jnp.sum(y, axis=0, keepdims=True)
        s2 = jnp.sum(y * y, axis=0, keepdims=True)
        part = jnp.concatenate([s1, s2, jnp.zeros((6, W), jnp.float32)], axis=0)

        @pl.when(i == 0)
        def _():
            st_ref[...] = part

        @pl.when(i > 0)
        def _():
            st_ref[...] += part

    return pl.pallas_call(
        body,
        grid=(NBLK,),
        in_specs=[pl.BlockSpec((BE, W), lambda i: (i, 0)),
                  pl.BlockSpec((BE, WD), lambda i: (i, 0)),
                  pl.BlockSpec((BE, W), lambda i: (i, 0)),
                  pl.BlockSpec((BE, 1), lambda i: (i, 0)),
                  pl.BlockSpec((W, W), lambda i: (0, 0)),
                  pl.BlockSpec((1, W), lambda i: (0, 0))],
        out_specs=(pl.BlockSpec((BE, W), lambda i: (i, 0)),
                   pl.BlockSpec((2, BE, W), lambda i: (0, i, 0)),
                   pl.BlockSpec((8, W), lambda i: (0, 0))),
        out_shape=(jax.ShapeDtypeStruct((E, W), jnp.float32),
                   jax.ShapeDtypeStruct((2, E, W), jnp.float32),
                   jax.ShapeDtypeStruct((8, W), jnp.float32)),
    )(e, gsrc, edst, enorm, cw, cb)


def _node_finish(ah, p, nnorm, h_in, gam, bet):
    def body(ah_ref, p_ref, nn_ref, h_ref, g_ref, b_ref, o_ref):
        den = p_ref[0]
        num = p_ref[1]
        hn = (ah_ref[...] + num / (den + 1e-6)) * nn_ref[...]
        mu = jnp.mean(hn, axis=0, keepdims=True)
        var = jnp.mean(hn * hn, axis=0, keepdims=True) - mu * mu
        bn = (hn - mu) / jnp.sqrt(var + 1e-5) * g_ref[...] + b_ref[...]
        o_ref[...] = h_ref[...] + jnp.maximum(bn, 0.0)
    return pl.pallas_call(
        body, out_shape=jax.ShapeDtypeStruct((N, W), jnp.float32),
    )(ah, p, nnorm, h_in, gam, bet)


def _edge_finish(y, e_in, st, gam, bet):
    def body(y_ref, e_ref, st_ref, g_ref, b_ref, o_ref):
        mu = st_ref[0:1, :] * (1.0 / E)
        var = st_ref[1:2, :] * (1.0 / E) - mu * mu
        bn = (y_ref[...] - mu) / jnp.sqrt(var + 1e-5) * g_ref[...] + b_ref[...]
        o_ref[...] = e_ref[...] + jnp.maximum(bn, 0.0)
    return pl.pallas_call(
        body,
        grid=(NBLK,),
        in_specs=[pl.BlockSpec((BE, W), lambda i: (i, 0)),
                  pl.BlockSpec((BE, W), lambda i: (i, 0)),
                  pl.BlockSpec((8, W), lambda i: (0, 0)),
                  pl.BlockSpec((1, W), lambda i: (0, 0)),
                  pl.BlockSpec((1, W), lambda i: (0, 0))],
        out_specs=pl.BlockSpec((BE, W), lambda i: (i, 0)),
        out_shape=jax.ShapeDtypeStruct((E, W), jnp.float32),
    )(y, e_in, st, gam, bet)


def _readout(h, w0, b0_, w1, b1_, w2, b2_):
    def body(h_ref, w0_ref, b0_ref, w1_ref, b1_ref, w2_ref, b2_ref, o_ref):
        hg = jnp.mean(h_ref[...], axis=0, keepdims=True)
        y0 = jnp.maximum(jnp.dot(hg, w0_ref[...],
                                 preferred_element_type=jnp.float32) + b0_ref[...], 0.0)
        y1 = jnp.maximum(jnp.dot(y0, w1_ref[...],
                                 preferred_element_type=jnp.float32) + b1_ref[...], 0.0)
        o_ref[...] = jnp.dot(y1, w2_ref[...],
                             preferred_element_type=jnp.float32) + b2_ref[...]
    return pl.pallas_call(
        body, out_shape=jax.ShapeDtypeStruct((1, 128), jnp.float32),
    )(h, w0, b0_, w1, b1_, w2, b2_)


def _padw(w, r, c):
    return jnp.zeros((r, c), jnp.float32).at[:w.shape[0], :w.shape[1]].set(w)


def _padb(b, c):
    return jnp.zeros((1, c), jnp.float32).at[0, :b.shape[0]].set(b)


def kernel(nodes_feat, edges_feat, nodes_num_norm_sqrt, edges_num_norm_sqrt, edge_index, Wh, bh, We, be, AW, Ab, BW, Bb, CW, Cb, DW, Db, EW, Eb, bn_h_gamma, bn_h_beta, bn_e_gamma, bn_e_beta, W0, b0, W1, b1, W2, b2):
    src3 = edge_index[0].reshape(NCHUNK, NST, 128)
    dst3 = edge_index[1].reshape(NCHUNK, NST, 128)

    h = _embed_h(nodes_feat, _padw(Wh, 128, W), _padb(bh, W))
    e = _embed_e(edges_feat, _padw(We, 1, W), _padb(be, W))
    zer = jnp.zeros((N, W), jnp.float32)

    for l in range(L):
        wc = jnp.concatenate([_padw(AW[l], W, W), _padw(BW[l], W, W),
                              _padw(DW[l], W, W), _padw(EW[l], W, W)], axis=1)
        bc = jnp.concatenate([_padb(Ab[l], W), _padb(Bb[l], W),
                              _padb(Db[l], W), _padb(Eb[l], W)], axis=1)
        ah, g, eh = _node_mm(h, wc, bc)
        gsrc, edst_ = _sc_gather(g, eh, src3, dst3)
        y, m, st = _edge_main(e, gsrc, edst_, edges_num_norm_sqrt,
                              _padw(CW[l], W, W), _padb(Cb[l], W))
        p = _sc_scatter(m, dst3, zer)
        h = _node_finish(ah, p, nodes_num_norm_sqrt, h,
                         _padb(bn_h_gamma[l], W), _padb(bn_h_beta[l], W))
        e = _edge_finish(y, e, st, _padb(bn_e_gamma[l], W), _padb(bn_e_beta[l], W))

    out = _readout(h, _padw(W0, W, 128), _padb(b0, 128),
                   _padw(W1, 128, 128), _padb(b1, 128),
                   _padw(W2, 128, 128), _padb(b2, 128))
    return out[0, :10]


# speedup vs baseline: 3.8581x; 1.1429x over previous
import functools

import jax
import jax.numpy as jnp
from jax import lax
from jax.experimental import pallas as pl
from jax.experimental.pallas import tpu as pltpu
from jax.experimental.pallas import tpu_sc as plsc

N = 10000
E = 320000
W = 128
WD = 256
L = 4
BE = 4000
NBLK = E // BE
NW = 32
CH = 1280
NCHUNK = E // CH
CPW = -(-NCHUNK // NW)
CPS = -(-NCHUNK // 16)
NST = CH // W


def _sc_gather(gtab, etab, src3, dst3):
    mesh = plsc.VectorSubcoreMesh(core_axis_name="c", subcore_axis_name="s")

    @functools.partial(
        pl.kernel,
        out_type=(jax.ShapeDtypeStruct((E, W), jnp.uint32),
                  jax.ShapeDtypeStruct((E, W), jnp.float32)),
        mesh=mesh,
        scratch_types=[
            pltpu.VMEM((NST, 128), jnp.int32),
            pltpu.VMEM((NST, 128), jnp.int32),
            pltpu.VMEM((2, 128, W), jnp.uint32),
            pltpu.VMEM((2, 128, W), jnp.float32),
            pltpu.SemaphoreType.DMA,
            pltpu.SemaphoreType.DMA,
            pltpu.SemaphoreType.DMA,
        ],
    )
    def k(gtab_hbm, etab_hbm, src_hbm, dst_hbm, gout_hbm, eout_hbm,
          sidx, didx, grows, erows, semg, seme, semw):
        wid = lax.axis_index("c") * 16 + lax.axis_index("s")

        @pl.loop(0, CPW)
        def _(i):
            c = wid + i * NW

            @pl.when(c < NCHUNK)
            def _():
                pltpu.sync_copy(src_hbm.at[c], sidx)
                pltpu.sync_copy(dst_hbm.at[c], didx)

                def gath(j):
                    b = j % 2
                    return (pltpu.async_copy(
                                gtab_hbm.at[sidx.at[j]], grows.at[b], semg),
                            pltpu.async_copy(
                                etab_hbm.at[didx.at[j]], erows.at[b], seme))

                def wr(j):
                    b = j % 2
                    base = c * CH + j * 128
                    return (pltpu.async_copy(
                                grows.at[b], gout_hbm.at[pl.ds(base, 128)], semw),
                            pltpu.async_copy(
                                erows.at[b], eout_hbm.at[pl.ds(base, 128)], semw))

                gcur = gath(0)
                wprev = None
                for j in range(NST):
                    for cp in gcur:
                        cp.wait()
                    if wprev is not None:
                        for cp in wprev:
                            cp.wait()
                    if j + 1 < NST:
                        gnext = gath(j + 1)
                    wcur = wr(j)
                    if j + 1 < NST:
                        gcur = gnext
                    wprev = wcur
                for cp in wprev:
                    cp.wait()

    return k(gtab, etab, src3, dst3)


def _sc_scatter(m, dst3, zer):
    mesh = plsc.VectorSubcoreMesh(core_axis_name="c", subcore_axis_name="s")

    @functools.partial(
        pl.kernel,
        out_type=jax.ShapeDtypeStruct((2, N, W), jnp.float32),
        mesh=mesh,
        scratch_types=[
            pltpu.VMEM((NST, 128), jnp.int32),
            pltpu.VMEM((2, 128, W), jnp.float32),
            pltpu.VMEM_SHARED((N, W), jnp.float32),
            pltpu.SemaphoreType.DMA,
            pltpu.SemaphoreType.DMA,
        ],
    )
    def k(m_hbm, dst_hbm, z_hbm, p_hbm, didx, mrows, accum, semld, semsc):
        core = lax.axis_index("c")
        s = lax.axis_index("s")

        @pl.when(s == 0)
        def _():
            pltpu.sync_copy(z_hbm, accum)

        plsc.subcore_barrier()

        @pl.loop(0, CPS)
        def _(i):
            c = s + i * 16

            @pl.when(c < NCHUNK)
            def _():
                pltpu.sync_copy(dst_hbm.at[c], didx)

                def mload(j):
                    return pltpu.async_copy(
                        m_hbm.at[core, pl.ds(c * CH + j * 128, 128)],
                        mrows.at[j % 2], semld)

                lcur = mload(0)
                scprev = None
                for j in range(NST):
                    lcur.wait()
                    if scprev is not None:
                        scprev.wait()
                    if j + 1 < NST:
                        lnext = mload(j + 1)
                    sccur = pltpu.async_copy(
                        mrows.at[j % 2], accum.at[didx.at[j]], semsc, add=True)
                    if j + 1 < NST:
                        lcur = lnext
                    scprev = sccur
                scprev.wait()

        plsc.subcore_barrier()

        @pl.when(s < 10)
        def _():
            pltpu.sync_copy(accum.at[pl.ds(s * 1000, 1000)],
                            p_hbm.at[core, pl.ds(s * 1000, 1000)])

    return k(m, dst3, zer)


def _embed_h(nodes_feat, whp, bhp):
    def body(x_ref, w_ref, b_ref, o_ref):
        o_ref[...] = jnp.dot(x_ref[...], w_ref[...],
                             preferred_element_type=jnp.float32) + b_ref[...]
    return pl.pallas_call(
        body, out_shape=jax.ShapeDtypeStruct((N, W), jnp.float32),
    )(nodes_feat, whp, bhp)


def _embed_e(ef, wep, bep):
    def body(f_ref, w_ref, b_ref, o_ref):
        o_ref[...] = f_ref[...] * w_ref[...] + b_ref[...]
    return pl.pallas_call(
        body,
        grid=(NBLK,),
        in_specs=[pl.BlockSpec((BE, 1), lambda i: (i, 0)),
                  pl.BlockSpec((1, W), lambda i: (0, 0)),
                  pl.BlockSpec((1, W), lambda i: (0, 0))],
        out_specs=pl.BlockSpec((BE, W), lambda i: (i, 0)),
        out_shape=jax.ShapeDtypeStruct((E, W), jnp.float32),
    )(ef, wep, bep)


def _node_mm(h, wc, bc):
    def body(h_ref, w_ref, b_ref, ah_ref, g_ref, eh_ref):
        hw = jnp.dot(h_ref[...], w_ref[...],
                     preferred_element_type=jnp.float32) + b_ref[...]
        ah_ref[...] = hw[:, 0:W]
        dhb = lax.bitcast_convert_type(
            hw[:, 2 * W:3 * W].astype(jnp.bfloat16).astype(jnp.float32),
            jnp.uint32)
        bhb = lax.bitcast_convert_type(
            hw[:, W:2 * W].astype(jnp.bfloat16).astype(jnp.float32),
            jnp.uint32)
        g_ref[...] = (dhb & jnp.uint32(0xFFFF0000)) | (bhb >> 16)
        eh_ref[...] = hw[:, 3 * W:4 * W]
    return pl.pallas_call(
        body,
        out_shape=(jax.ShapeDtypeStruct((N, W), jnp.float32),
                   jax.ShapeDtypeStruct((N, W), jnp.uint32),
                   jax.ShapeDtypeStruct((N, W), jnp.float32)),
    )(h, wc, bc)


def _edge_main(e, gsrc, edst, enorm, cw, cb):
    def body(e_ref, g_ref, ed_ref, n_ref, w_ref, b_ref, y_ref, m_ref, st_ref):
        i = pl.program_id(0)
        ce = jnp.dot(e_ref[...], w_ref[...],
                     preferred_element_type=jnp.float32) + b_ref[...]
        gu = g_ref[...]
        d = lax.bitcast_convert_type(gu & jnp.uint32(0xFFFF0000), jnp.float32)
        bmsg = lax.bitcast_convert_type(gu << 16, jnp.float32)
        x = ce + d + ed_ref[...]
        sig = jax.nn.sigmoid(x)
        y = x * n_ref[...]
        y_ref[...] = y
        m_ref[0] = sig
        m_ref[1] = sig * bmsg
        s1 = jnp.sum(y, axis=0, keepdims=True)
        s2 = jnp.sum(y * y, axis=0, keepdims=True)
        part = jnp.concatenate([s1, s2, jnp.zeros((6, W), jnp.float32)], axis=0)

        @pl.when(i == 0)
        def _():
            st_ref[...] = part

        @pl.when(i > 0)
        def _():
            st_ref[...] += part

    return pl.pallas_call(
        body,
        grid=(NBLK,),
        in_specs=[pl.BlockSpec((BE, W), lambda i: (i, 0)),
                  pl.BlockSpec((BE, W), lambda i: (i, 0)),
                  pl.BlockSpec((BE, W), lambda i: (i, 0)),
                  pl.BlockSpec((BE, 1), lambda i: (i, 0)),
                  pl.BlockSpec((W, W), lambda i: (0, 0)),
                  pl.BlockSpec((1, W), lambda i: (0, 0))],
        out_specs=(pl.BlockSpec((BE, W), lambda i: (i, 0)),
                   pl.BlockSpec((2, BE, W), lambda i: (0, i, 0)),
                   pl.BlockSpec((8, W), lambda i: (0, 0))),
        out_shape=(jax.ShapeDtypeStruct((E, W), jnp.float32),
                   jax.ShapeDtypeStruct((2, E, W), jnp.float32),
                   jax.ShapeDtypeStruct((8, W), jnp.float32)),
    )(e, gsrc, edst, enorm, cw, cb)


def _node_finish(ah, p, nnorm, h_in, gam, bet):
    def body(ah_ref, p_ref, nn_ref, h_ref, g_ref, b_ref, o_ref):
        den = p_ref[0]
        num = p_ref[1]
        hn = (ah_ref[...] + num / (den + 1e-6)) * nn_ref[...]
        mu = jnp.mean(hn, axis=0, keepdims=True)
        var = jnp.mean(hn * hn, axis=0, keepdims=True) - mu * mu
        bn = (hn - mu) / jnp.sqrt(var + 1e-5) * g_ref[...] + b_ref[...]
        o_ref[...] = h_ref[...] + jnp.maximum(bn, 0.0)
    return pl.pallas_call(
        body, out_shape=jax.ShapeDtypeStruct((N, W), jnp.float32),
    )(ah, p, nnorm, h_in, gam, bet)


def _edge_finish(y, e_in, st, gam, bet):
    def body(y_ref, e_ref, st_ref, g_ref, b_ref, o_ref):
        mu = st_ref[0:1, :] * (1.0 / E)
        var = st_ref[1:2, :] * (1.0 / E) - mu * mu
        bn = (y_ref[...] - mu) / jnp.sqrt(var + 1e-5) * g_ref[...] + b_ref[...]
        o_ref[...] = e_ref[...] + jnp.maximum(bn, 0.0)
    return pl.pallas_call(
        body,
        grid=(NBLK,),
        in_specs=[pl.BlockSpec((BE, W), lambda i: (i, 0)),
                  pl.BlockSpec((BE, W), lambda i: (i, 0)),
                  pl.BlockSpec((8, W), lambda i: (0, 0)),
                  pl.BlockSpec((1, W), lambda i: (0, 0)),
                  pl.BlockSpec((1, W), lambda i: (0, 0))],
        out_specs=pl.BlockSpec((BE, W), lambda i: (i, 0)),
        out_shape=jax.ShapeDtypeStruct((E, W), jnp.float32),
    )(y, e_in, st, gam, bet)


def _readout(h, w0, b0_, w1, b1_, w2, b2_):
    def body(h_ref, w0_ref, b0_ref, w1_ref, b1_ref, w2_ref, b2_ref, o_ref):
        hg = jnp.mean(h_ref[...], axis=0, keepdims=True)
        y0 = jnp.maximum(jnp.dot(hg, w0_ref[...],
                                 preferred_element_type=jnp.float32) + b0_ref[...], 0.0)
        y1 = jnp.maximum(jnp.dot(y0, w1_ref[...],
                                 preferred_element_type=jnp.float32) + b1_ref[...], 0.0)
        o_ref[...] = jnp.dot(y1, w2_ref[...],
                             preferred_element_type=jnp.float32) + b2_ref[...]
    return pl.pallas_call(
        body, out_shape=jax.ShapeDtypeStruct((1, 128), jnp.float32),
    )(h, w0, b0_, w1, b1_, w2, b2_)


def _padw(w, r, c):
    return jnp.zeros((r, c), jnp.float32).at[:w.shape[0], :w.shape[1]].set(w)


def _padb(b, c):
    return jnp.zeros((1, c), jnp.float32).at[0, :b.shape[0]].set(b)


def kernel(nodes_feat, edges_feat, nodes_num_norm_sqrt, edges_num_norm_sqrt, edge_index, Wh, bh, We, be, AW, Ab, BW, Bb, CW, Cb, DW, Db, EW, Eb, bn_h_gamma, bn_h_beta, bn_e_gamma, bn_e_beta, W0, b0, W1, b1, W2, b2):
    src3 = edge_index[0].reshape(NCHUNK, NST, 128)
    dst3 = edge_index[1].reshape(NCHUNK, NST, 128)

    h = _embed_h(nodes_feat, _padw(Wh, 128, W), _padb(bh, W))
    e = _embed_e(edges_feat, _padw(We, 1, W), _padb(be, W))
    zer = jnp.zeros((N, W), jnp.float32)

    for l in range(L):
        wc = jnp.concatenate([_padw(AW[l], W, W), _padw(BW[l], W, W),
                              _padw(DW[l], W, W), _padw(EW[l], W, W)], axis=1)
        bc = jnp.concatenate([_padb(Ab[l], W), _padb(Bb[l], W),
                              _padb(Db[l], W), _padb(Eb[l], W)], axis=1)
        ah, g, eh = _node_mm(h, wc, bc)
        gsrc, edst_ = _sc_gather(g, eh, src3, dst3)
        y, m, st = _edge_main(e, gsrc, edst_, edges_num_norm_sqrt,
                              _padw(CW[l], W, W), _padb(Cb[l], W))
        p = _sc_scatter(m, dst3, zer)
        h = _node_finish(ah, p, nodes_num_norm_sqrt, h,
                         _padb(bn_h_gamma[l], W), _padb(bn_h_beta[l], W))
        e = _edge_finish(y, e, st, _padb(bn_e_gamma[l], W), _padb(bn_e_beta[l], W))

    out = _readout(h, _padw(W0, W, 128), _padb(b0, 128),
                   _padw(W1, 128, 128), _padb(b1, 128),
                   _padw(W2, 128, 128), _padb(b2, 128))
    return out[0, :10]


# speedup vs baseline: 4.2754x; 1.1082x over previous
import functools

import jax
import jax.numpy as jnp
from jax import lax
from jax.experimental import pallas as pl
from jax.experimental.pallas import tpu as pltpu
from jax.experimental.pallas import tpu_sc as plsc

N = 10000
E = 320000
W = 128
WD = 256
L = 4
BE = 4000
NBLK = E // BE
NW = 32
CH = 1280
NCHUNK = E // CH
CPW = -(-NCHUNK // NW)
CPS = -(-NCHUNK // 16)
NST = CH // W


def _sc_gather(gtab, etab, src3, dst3):
    mesh = plsc.VectorSubcoreMesh(core_axis_name="c", subcore_axis_name="s")

    @functools.partial(
        pl.kernel,
        out_type=(jax.ShapeDtypeStruct((E, W), jnp.uint32),
                  jax.ShapeDtypeStruct((E, W), jnp.float32)),
        mesh=mesh,
        scratch_types=[
            pltpu.VMEM((NST, 128), jnp.int32),
            pltpu.VMEM((NST, 128), jnp.int32),
            pltpu.VMEM((2, 128, W), jnp.uint32),
            pltpu.VMEM((2, 128, W), jnp.float32),
            pltpu.SemaphoreType.DMA,
            pltpu.SemaphoreType.DMA,
            pltpu.SemaphoreType.DMA,
        ],
    )
    def k(gtab_hbm, etab_hbm, src_hbm, dst_hbm, gout_hbm, eout_hbm,
          sidx, didx, grows, erows, semg, seme, semw):
        wid = lax.axis_index("c") * 16 + lax.axis_index("s")

        @pl.loop(0, CPW)
        def _(i):
            c = wid + i * NW

            @pl.when(c < NCHUNK)
            def _():
                pltpu.sync_copy(src_hbm.at[c], sidx)
                pltpu.sync_copy(dst_hbm.at[c], didx)

                def gath(j):
                    b = j % 2
                    return (pltpu.async_copy(
                                gtab_hbm.at[sidx.at[j]], grows.at[b], semg),
                            pltpu.async_copy(
                                etab_hbm.at[didx.at[j]], erows.at[b], seme))

                def wr(j):
                    b = j % 2
                    base = c * CH + j * 128
                    return (pltpu.async_copy(
                                grows.at[b], gout_hbm.at[pl.ds(base, 128)], semw),
                            pltpu.async_copy(
                                erows.at[b], eout_hbm.at[pl.ds(base, 128)], semw))

                gcur = gath(0)
                wprev = None
                for j in range(NST):
                    for cp in gcur:
                        cp.wait()
                    if wprev is not None:
                        for cp in wprev:
                            cp.wait()
                    if j + 1 < NST:
                        gnext = gath(j + 1)
                    wcur = wr(j)
                    if j + 1 < NST:
                        gcur = gnext
                    wprev = wcur
                for cp in wprev:
                    cp.wait()

    return k(gtab, etab, src3, dst3)


def _sc_scatter(m, dst3, zer):
    mesh = plsc.VectorSubcoreMesh(core_axis_name="c", subcore_axis_name="s")

    @functools.partial(
        pl.kernel,
        out_type=jax.ShapeDtypeStruct((2, N, W), jnp.float32),
        mesh=mesh,
        scratch_types=[
            pltpu.VMEM((NST, 128), jnp.int32),
            pltpu.VMEM((2, 128, W), jnp.float32),
            pltpu.VMEM_SHARED((N, W), jnp.float32),
            pltpu.SemaphoreType.DMA,
            pltpu.SemaphoreType.DMA,
        ],
    )
    def k(m_hbm, dst_hbm, z_hbm, p_hbm, didx, mrows, accum, semld, semsc):
        core = lax.axis_index("c")
        s = lax.axis_index("s")

        @pl.when(s == 0)
        def _():
            pltpu.sync_copy(z_hbm, accum)

        plsc.subcore_barrier()

        @pl.loop(0, CPS)
        def _(i):
            c = s + i * 16

            @pl.when(c < NCHUNK)
            def _():
                pltpu.sync_copy(dst_hbm.at[c], didx)

                def mload(j):
                    return pltpu.async_copy(
                        m_hbm.at[core, pl.ds(c * CH + j * 128, 128)],
                        mrows.at[j % 2], semld)

                lcur = mload(0)
                scprev = None
                for j in range(NST):
                    lcur.wait()
                    if scprev is not None:
                        scprev.wait()
                    if j + 1 < NST:
                        lnext = mload(j + 1)
                    sccur = pltpu.async_copy(
                        mrows.at[j % 2], accum.at[didx.at[j]], semsc, add=True)
                    if j + 1 < NST:
                        lcur = lnext
                    scprev = sccur
                scprev.wait()

        plsc.subcore_barrier()

        @pl.when(s < 10)
        def _():
            pltpu.sync_copy(accum.at[pl.ds(s * 1000, 1000)],
                            p_hbm.at[core, pl.ds(s * 1000, 1000)])

    return k(m, dst3, zer)


def _embed_h(nodes_feat, whp, bhp):
    def body(x_ref, w_ref, b_ref, o_ref):
        o_ref[...] = jnp.dot(x_ref[...], w_ref[...],
                             preferred_element_type=jnp.float32) + b_ref[...]
    return pl.pallas_call(
        body, out_shape=jax.ShapeDtypeStruct((N, W), jnp.float32),
    )(nodes_feat, whp, bhp)


def _node_mm(h, wc, bc):
    def body(h_ref, w_ref, b_ref, ah_ref, g_ref, eh_ref):
        hw = jnp.dot(h_ref[...], w_ref[...],
                     preferred_element_type=jnp.float32) + b_ref[...]
        ah_ref[...] = hw[:, 0:W]
        dhb = lax.bitcast_convert_type(
            hw[:, 2 * W:3 * W].astype(jnp.bfloat16).astype(jnp.float32),
            jnp.uint32)
        bhb = lax.bitcast_convert_type(
            hw[:, W:2 * W].astype(jnp.bfloat16).astype(jnp.float32),
            jnp.uint32)
        g_ref[...] = (dhb & jnp.uint32(0xFFFF0000)) | (bhb >> 16)
        eh_ref[...] = hw[:, 3 * W:4 * W]
    return pl.pallas_call(
        body,
        out_shape=(jax.ShapeDtypeStruct((N, W), jnp.float32),
                   jax.ShapeDtypeStruct((N, W), jnp.uint32),
                   jax.ShapeDtypeStruct((N, W), jnp.float32)),
    )(h, wc, bc)


def _edge_main(l, args):
    def body(*refs):
        it = iter(refs)
        if l == 0:
            f_ref = next(it)
        elif l == 1:
            f_ref, yp_ref, stp_ref, gp_ref, bp_ref = (next(it) for _ in range(5))
        else:
            ep_ref, yp_ref, stp_ref, gp_ref, bp_ref = (next(it) for _ in range(5))
        g_ref, ed_ref = next(it), next(it)
        if l < 3:
            n_ref = next(it)
        w_ref, b_ref = next(it), next(it)
        if l <= 1:
            we_ref, be_ref = next(it), next(it)
        if l in (1, 2):
            eo_ref = next(it)
        if l < 3:
            y_ref = next(it)
        m_ref = next(it)
        if l < 3:
            st_ref = next(it)

        if l == 0:
            e_in = f_ref[...] * we_ref[...] + be_ref[...]
        else:
            if l == 1:
                e_base = f_ref[...] * we_ref[...] + be_ref[...]
            else:
                e_base = ep_ref[...]
            mu = stp_ref[0:1, :] * (1.0 / E)
            var = stp_ref[1:2, :] * (1.0 / E) - mu * mu
            bn = ((yp_ref[...] - mu) / jnp.sqrt(var + 1e-5) * gp_ref[...]
                  + bp_ref[...])
            e_in = e_base + jnp.maximum(bn, 0.0)
        if l in (1, 2):
            eo_ref[...] = e_in

        i = pl.program_id(0)
        ce = jnp.dot(e_in, w_ref[...],
                     preferred_element_type=jnp.float32) + b_ref[...]
        gu = g_ref[...]
        d = lax.bitcast_convert_type(gu & jnp.uint32(0xFFFF0000), jnp.float32)
        bmsg = lax.bitcast_convert_type(gu << 16, jnp.float32)
        x = ce + d + ed_ref[...]
        sig = jax.nn.sigmoid(x)
        m_ref[0] = sig
        m_ref[1] = sig * bmsg
        if l < 3:
            y = x * n_ref[...]
            y_ref[...] = y
            s1 = jnp.sum(y, axis=0, keepdims=True)
            s2 = jnp.sum(y * y, axis=0, keepdims=True)
            part = jnp.concatenate(
                [s1, s2, jnp.zeros((6, W), jnp.float32)], axis=0)

            @pl.when(i == 0)
            def _():
                st_ref[...] = part

            @pl.when(i > 0)
            def _():
                st_ref[...] += part

    eblk = lambda w=W: pl.BlockSpec((BE, w), lambda i: (i, 0))
    cblk = lambda r: pl.BlockSpec((r, W), lambda i: (0, 0))
    in_specs = []
    if l == 0:
        in_specs += [pl.BlockSpec((BE, 1), lambda i: (i, 0))]
    elif l == 1:
        in_specs += [pl.BlockSpec((BE, 1), lambda i: (i, 0)),
                     eblk(), cblk(8), cblk(1), cblk(1)]
    else:
        in_specs += [eblk(), eblk(), cblk(8), cblk(1), cblk(1)]
    in_specs += [eblk(), eblk()]
    if l < 3:
        in_specs += [pl.BlockSpec((BE, 1), lambda i: (i, 0))]
    in_specs += [cblk(W), cblk(1)]
    if l <= 1:
        in_specs += [cblk(1), cblk(1)]

    out_specs, out_shape = [], []
    if l in (1, 2):
        out_specs += [eblk()]
        out_shape += [jax.ShapeDtypeStruct((E, W), jnp.float32)]
    if l < 3:
        out_specs += [eblk()]
        out_shape += [jax.ShapeDtypeStruct((E, W), jnp.float32)]
    out_specs += [pl.BlockSpec((2, BE, W), lambda i: (0, i, 0))]
    out_shape += [jax.ShapeDtypeStruct((2, E, W), jnp.float32)]
    if l < 3:
        out_specs += [cblk(8)]
        out_shape += [jax.ShapeDtypeStruct((8, W), jnp.float32)]

    return pl.pallas_call(
        body,
        grid=(NBLK,),
        in_specs=in_specs,
        out_specs=tuple(out_specs),
        out_shape=tuple(out_shape),
    )(*args)


def _node_finish(ah, p, nnorm, h_in, gam, bet):
    def body(ah_ref, p_ref, nn_ref, h_ref, g_ref, b_ref, o_ref):
        den = p_ref[0]
        num = p_ref[1]
        hn = (ah_ref[...] + num / (den + 1e-6)) * nn_ref[...]
        mu = jnp.mean(hn, axis=0, keepdims=True)
        var = jnp.mean(hn * hn, axis=0, keepdims=True) - mu * mu
        bn = (hn - mu) / jnp.sqrt(var + 1e-5) * g_ref[...] + b_ref[...]
        o_ref[...] = h_ref[...] + jnp.maximum(bn, 0.0)
    return pl.pallas_call(
        body, out_shape=jax.ShapeDtypeStruct((N, W), jnp.float32),
    )(ah, p, nnorm, h_in, gam, bet)


def _readout(h, w0, b0_, w1, b1_, w2, b2_):
    def body(h_ref, w0_ref, b0_ref, w1_ref, b1_ref, w2_ref, b2_ref, o_ref):
        hg = jnp.mean(h_ref[...], axis=0, keepdims=True)
        y0 = jnp.maximum(jnp.dot(hg, w0_ref[...],
                                 preferred_element_type=jnp.float32) + b0_ref[...], 0.0)
        y1 = jnp.maximum(jnp.dot(y0, w1_ref[...],
                                 preferred_element_type=jnp.float32) + b1_ref[...], 0.0)
        o_ref[...] = jnp.dot(y1, w2_ref[...],
                             preferred_element_type=jnp.float32) + b2_ref[...]
    return pl.pallas_call(
        body, out_shape=jax.ShapeDtypeStruct((1, 128), jnp.float32),
    )(h, w0, b0_, w1, b1_, w2, b2_)


def _padw(w, r, c):
    return jnp.zeros((r, c), jnp.float32).at[:w.shape[0], :w.shape[1]].set(w)


def _padb(b, c):
    return jnp.zeros((1, c), jnp.float32).at[0, :b.shape[0]].set(b)


def kernel(nodes_feat, edges_feat, nodes_num_norm_sqrt, edges_num_norm_sqrt, edge_index, Wh, bh, We, be, AW, Ab, BW, Bb, CW, Cb, DW, Db, EW, Eb, bn_h_gamma, bn_h_beta, bn_e_gamma, bn_e_beta, W0, b0, W1, b1, W2, b2):
    src3 = edge_index[0].reshape(NCHUNK, NST, 128)
    dst3 = edge_index[1].reshape(NCHUNK, NST, 128)

    h = _embed_h(nodes_feat, _padw(Wh, 128, W), _padb(bh, W))
    zer = jnp.zeros((N, W), jnp.float32)
    wep = _padw(We, 1, W)
    bep = _padb(be, W)

    y_prev = st_prev = e_prev = None
    for l in range(L):
        wc = jnp.concatenate([_padw(AW[l], W, W), _padw(BW[l], W, W),
                              _padw(DW[l], W, W), _padw(EW[l], W, W)], axis=1)
        bc = jnp.concatenate([_padb(Ab[l], W), _padb(Bb[l], W),
                              _padb(Db[l], W), _padb(Eb[l], W)], axis=1)
        ah, g, eh = _node_mm(h, wc, bc)
        gsrc, edst_ = _sc_gather(g, eh, src3, dst3)
        cw = _padw(CW[l], W, W)
        cb = _padb(Cb[l], W)
        if l == 0:
            args = [edges_feat, gsrc, edst_, edges_num_norm_sqrt, cw, cb,
                    wep, bep]
            y_prev, m, st_prev = _edge_main(0, args)
        elif l == 1:
            args = [edges_feat, y_prev, st_prev,
                    _padb(bn_e_gamma[0], W), _padb(bn_e_beta[0], W),
                    gsrc, edst_, edges_num_norm_sqrt, cw, cb, wep, bep]
            e_prev, y_prev, m, st_prev = _edge_main(1, args)
        elif l == 2:
            args = [e_prev, y_prev, st_prev,
                    _padb(bn_e_gamma[1], W), _padb(bn_e_beta[1], W),
                    gsrc, edst_, edges_num_norm_sqrt, cw, cb]
            e_prev, y_prev, m, st_prev = _edge_main(2, args)
        else:
            args = [e_prev, y_prev, st_prev,
                    _padb(bn_e_gamma[2], W), _padb(bn_e_beta[2], W),
                    gsrc, edst_, cw, cb]
            (m,) = _edge_main(3, args)
        p = _sc_scatter(m, dst3, zer)
        h = _node_finish(ah, p, nodes_num_norm_sqrt, h,
                         _padb(bn_h_gamma[l], W), _padb(bn_h_beta[l], W))

    out = _readout(h, _padw(W0, W, 128), _padb(b0, 128),
                   _padw(W1, 128, 128), _padb(b1, 128),
                   _padw(W2, 128, 128), _padb(b2, 128))
    return out[0, :10]


# speedup vs baseline: 4.7507x; 1.1112x over previous
import functools

import jax
import jax.numpy as jnp
from jax import lax
from jax.experimental import pallas as pl
from jax.experimental.pallas import tpu as pltpu
from jax.experimental.pallas import tpu_sc as plsc

N = 10000
E = 320000
W = 128
WD = 256
L = 4
BE = 4000
NBLK = E // BE
NW = 32
CH = 1280
NCHUNK = E // CH
CPW = -(-NCHUNK // NW)
CPS = -(-NCHUNK // 16)
NST = CH // W


def _sc_gather(tab, idx2):
    mesh = plsc.VectorSubcoreMesh(core_axis_name="c", subcore_axis_name="s")

    @functools.partial(
        pl.kernel,
        out_type=jax.ShapeDtypeStruct((2, E, W), jnp.uint32),
        mesh=mesh,
        scratch_types=[
            pltpu.VMEM((NST, 128), jnp.int32),
            pltpu.VMEM((2, 128, W), jnp.uint32),
            pltpu.VMEM_SHARED((N, W), jnp.uint32),
            pltpu.SemaphoreType.DMA,
            pltpu.SemaphoreType.DMA,
        ],
    )
    def k(tab_hbm, idx_hbm, out_hbm, idxv, rows, stab, semg, semw):
        core = lax.axis_index("c")
        s = lax.axis_index("s")

        @pl.when(s < 10)
        def _():
            pltpu.sync_copy(tab_hbm.at[core, pl.ds(s * 1000, 1000)],
                            stab.at[pl.ds(s * 1000, 1000)])

        plsc.subcore_barrier()

        @pl.loop(0, CPS)
        def _(i):
            c = s + i * 16

            @pl.when(c < NCHUNK)
            def _():
                pltpu.sync_copy(idx_hbm.at[core, c], idxv)

                def gath(j):
                    return pltpu.async_copy(
                        stab.at[idxv.at[j]], rows.at[j % 2], semg)

                def wr(j):
                    base = c * CH + j * 128
                    return pltpu.async_copy(
                        rows.at[j % 2], out_hbm.at[core, pl.ds(base, 128)],
                        semw)

                gcur = gath(0)
                wprev = None
                for j in range(NST):
                    gcur.wait()
                    if wprev is not None:
                        wprev.wait()
                    if j + 1 < NST:
                        gnext = gath(j + 1)
                    wcur = wr(j)
                    if j + 1 < NST:
                        gcur = gnext
                    wprev = wcur
                wprev.wait()

    return k(tab, idx2)


def _sc_scatter(m, dst3, zer):
    mesh = plsc.VectorSubcoreMesh(core_axis_name="c", subcore_axis_name="s")

    @functools.partial(
        pl.kernel,
        out_type=jax.ShapeDtypeStruct((2, N, W), jnp.float32),
        mesh=mesh,
        scratch_types=[
            pltpu.VMEM((NST, 128), jnp.int32),
            pltpu.VMEM((2, 128, W), jnp.float32),
            pltpu.VMEM_SHARED((N, W), jnp.float32),
            pltpu.SemaphoreType.DMA,
            pltpu.SemaphoreType.DMA,
        ],
    )
    def k(m_hbm, dst_hbm, z_hbm, p_hbm, didx, mrows, accum, semld, semsc):
        core = lax.axis_index("c")
        s = lax.axis_index("s")

        @pl.when(s == 0)
        def _():
            pltpu.sync_copy(z_hbm, accum)

        plsc.subcore_barrier()

        @pl.loop(0, CPS)
        def _(i):
            c = s + i * 16

            @pl.when(c < NCHUNK)
            def _():
                pltpu.sync_copy(dst_hbm.at[c], didx)

                def mload(j):
                    return pltpu.async_copy(
                        m_hbm.at[core, pl.ds(c * CH + j * 128, 128)],
                        mrows.at[j % 2], semld)

                lcur = mload(0)
                scprev = None
                for j in range(NST):
                    lcur.wait()
                    if scprev is not None:
                        scprev.wait()
                    if j + 1 < NST:
                        lnext = mload(j + 1)
                    sccur = pltpu.async_copy(
                        mrows.at[j % 2], accum.at[didx.at[j]], semsc, add=True)
                    if j + 1 < NST:
                        lcur = lnext
                    scprev = sccur
                scprev.wait()

        plsc.subcore_barrier()

        @pl.when(s < 10)
        def _():
            pltpu.sync_copy(accum.at[pl.ds(s * 1000, 1000)],
                            p_hbm.at[core, pl.ds(s * 1000, 1000)])

    return k(m, dst3, zer)


def _embed_h(nodes_feat, whp, bhp):
    def body(x_ref, w_ref, b_ref, o_ref):
        o_ref[...] = jnp.dot(x_ref[...], w_ref[...],
                             preferred_element_type=jnp.float32) + b_ref[...]
    return pl.pallas_call(
        body, out_shape=jax.ShapeDtypeStruct((N, W), jnp.float32),
    )(nodes_feat, whp, bhp)


def _node_mm(h, wc, bc):
    def body(h_ref, w_ref, b_ref, ah_ref, t_ref):
        hw = jnp.dot(h_ref[...], w_ref[...],
                     preferred_element_type=jnp.float32) + b_ref[...]
        ah_ref[...] = hw[:, 0:W]
        dhb = lax.bitcast_convert_type(
            hw[:, 2 * W:3 * W].astype(jnp.bfloat16).astype(jnp.float32),
            jnp.uint32)
        bhb = lax.bitcast_convert_type(
            hw[:, W:2 * W].astype(jnp.bfloat16).astype(jnp.float32),
            jnp.uint32)
        t_ref[0] = (dhb & jnp.uint32(0xFFFF0000)) | (bhb >> 16)
        t_ref[1] = lax.bitcast_convert_type(hw[:, 3 * W:4 * W], jnp.uint32)
    return pl.pallas_call(
        body,
        out_shape=(jax.ShapeDtypeStruct((N, W), jnp.float32),
                   jax.ShapeDtypeStruct((2, N, W), jnp.uint32)),
    )(h, wc, bc)


def _edge_main(l, args):
    def body(*refs):
        it = iter(refs)
        if l == 0:
            f_ref = next(it)
        elif l == 1:
            f_ref, yp_ref, stp_ref, gp_ref, bp_ref = (next(it) for _ in range(5))
        else:
            ep_ref, yp_ref, stp_ref, gp_ref, bp_ref = (next(it) for _ in range(5))
        g_ref, ed_ref = next(it), next(it)
        if l < 3:
            n_ref = next(it)
        w_ref, b_ref = next(it), next(it)
        if l <= 1:
            we_ref, be_ref = next(it), next(it)
        if l in (1, 2):
            eo_ref = next(it)
        if l < 3:
            y_ref = next(it)
        m_ref = next(it)
        if l < 3:
            st_ref = next(it)

        if l == 0:
            e_in = f_ref[...] * we_ref[...] + be_ref[...]
        else:
            if l == 1:
                e_base = f_ref[...] * we_ref[...] + be_ref[...]
            else:
                e_base = ep_ref[...]
            mu = stp_ref[0:1, :] * (1.0 / E)
            var = stp_ref[1:2, :] * (1.0 / E) - mu * mu
            bn = ((yp_ref[...] - mu) / jnp.sqrt(var + 1e-5) * gp_ref[...]
                  + bp_ref[...])
            e_in = e_base + jnp.maximum(bn, 0.0)
        if l in (1, 2):
            eo_ref[...] = e_in

        i = pl.program_id(0)
        ce = jnp.dot(e_in, w_ref[...],
                     preferred_element_type=jnp.float32) + b_ref[...]
        gu = g_ref[0]
        d = lax.bitcast_convert_type(gu & jnp.uint32(0xFFFF0000), jnp.float32)
        bmsg = lax.bitcast_convert_type(gu << 16, jnp.float32)
        x = ce + d + lax.bitcast_convert_type(ed_ref[0], jnp.float32)
        sig = jax.nn.sigmoid(x)
        m_ref[0] = sig
        m_ref[1] = sig * bmsg
        if l < 3:
            y = x * n_ref[...]
            y_ref[...] = y
            s1 = jnp.sum(y, axis=0, keepdims=True)
            s2 = jnp.sum(y * y, axis=0, keepdims=True)
            part = jnp.concatenate(
                [s1, s2, jnp.zeros((6, W), jnp.float32)], axis=0)

            @pl.when(i == 0)
            def _():
                st_ref[...] = part

            @pl.when(i > 0)
            def _():
                st_ref[...] += part

    eblk = lambda w=W: pl.BlockSpec((BE, w), lambda i: (i, 0))
    cblk = lambda r: pl.BlockSpec((r, W), lambda i: (0, 0))
    in_specs = []
    if l == 0:
        in_specs += [pl.BlockSpec((BE, 1), lambda i: (i, 0))]
    elif l == 1:
        in_specs += [pl.BlockSpec((BE, 1), lambda i: (i, 0)),
                     eblk(), cblk(8), cblk(1), cblk(1)]
    else:
        in_specs += [eblk(), eblk(), cblk(8), cblk(1), cblk(1)]
    in_specs += [pl.BlockSpec((1, BE, W), lambda i: (0, i, 0)),
                 pl.BlockSpec((1, BE, W), lambda i: (1, i, 0))]
    if l < 3:
        in_specs += [pl.BlockSpec((BE, 1), lambda i: (i, 0))]
    in_specs += [cblk(W), cblk(1)]
    if l <= 1:
        in_specs += [cblk(1), cblk(1)]

    out_specs, out_shape = [], []
    if l in (1, 2):
        out_specs += [eblk()]
        out_shape += [jax.ShapeDtypeStruct((E, W), jnp.float32)]
    if l < 3:
        out_specs += [eblk()]
        out_shape += [jax.ShapeDtypeStruct((E, W), jnp.float32)]
    out_specs += [pl.BlockSpec((2, BE, W), lambda i: (0, i, 0))]
    out_shape += [jax.ShapeDtypeStruct((2, E, W), jnp.float32)]
    if l < 3:
        out_specs += [cblk(8)]
        out_shape += [jax.ShapeDtypeStruct((8, W), jnp.float32)]

    return pl.pallas_call(
        body,
        grid=(NBLK,),
        in_specs=in_specs,
        out_specs=tuple(out_specs),
        out_shape=tuple(out_shape),
    )(*args)


def _node_finish(ah, p, nnorm, h_in, gam, bet):
    def body(ah_ref, p_ref, nn_ref, h_ref, g_ref, b_ref, o_ref):
        den = p_ref[0]
        num = p_ref[1]
        hn = (ah_ref[...] + num / (den + 1e-6)) * nn_ref[...]
        mu = jnp.mean(hn, axis=0, keepdims=True)
        var = jnp.mean(hn * hn, axis=0, keepdims=True) - mu * mu
        bn = (hn - mu) / jnp.sqrt(var + 1e-5) * g_ref[...] + b_ref[...]
        o_ref[...] = h_ref[...] + jnp.maximum(bn, 0.0)
    return pl.pallas_call(
        body, out_shape=jax.ShapeDtypeStruct((N, W), jnp.float32),
    )(ah, p, nnorm, h_in, gam, bet)


def _readout(h, w0, b0_, w1, b1_, w2, b2_):
    def body(h_ref, w0_ref, b0_ref, w1_ref, b1_ref, w2_ref, b2_ref, o_ref):
        hg = jnp.mean(h_ref[...], axis=0, keepdims=True)
        y0 = jnp.maximum(jnp.dot(hg, w0_ref[...],
                                 preferred_element_type=jnp.float32) + b0_ref[...], 0.0)
        y1 = jnp.maximum(jnp.dot(y0, w1_ref[...],
                                 preferred_element_type=jnp.float32) + b1_ref[...], 0.0)
        o_ref[...] = jnp.dot(y1, w2_ref[...],
                             preferred_element_type=jnp.float32) + b2_ref[...]
    return pl.pallas_call(
        body, out_shape=jax.ShapeDtypeStruct((1, 128), jnp.float32),
    )(h, w0, b0_, w1, b1_, w2, b2_)


def _padw(w, r, c):
    return jnp.zeros((r, c), jnp.float32).at[:w.shape[0], :w.shape[1]].set(w)


def _padb(b, c):
    return jnp.zeros((1, c), jnp.float32).at[0, :b.shape[0]].set(b)


def kernel(nodes_feat, edges_feat, nodes_num_norm_sqrt, edges_num_norm_sqrt, edge_index, Wh, bh, We, be, AW, Ab, BW, Bb, CW, Cb, DW, Db, EW, Eb, bn_h_gamma, bn_h_beta, bn_e_gamma, bn_e_beta, W0, b0, W1, b1, W2, b2):
    idx2 = edge_index.reshape(2, NCHUNK, NST, 128)
    dst3 = edge_index[1].reshape(NCHUNK, NST, 128)

    h = _embed_h(nodes_feat, _padw(Wh, 128, W), _padb(bh, W))
    zer = jnp.zeros((N, W), jnp.float32)
    wep = _padw(We, 1, W)
    bep = _padb(be, W)

    y_prev = st_prev = e_prev = None
    for l in range(L):
        wc = jnp.concatenate([_padw(AW[l], W, W), _padw(BW[l], W, W),
                              _padw(DW[l], W, W), _padw(EW[l], W, W)], axis=1)
        bc = jnp.concatenate([_padb(Ab[l], W), _padb(Bb[l], W),
                              _padb(Db[l], W), _padb(Eb[l], W)], axis=1)
        ah, tab = _node_mm(h, wc, bc)
        gout = _sc_gather(tab, idx2)
        gsrc = edst_ = gout
        cw = _padw(CW[l], W, W)
        cb = _padb(Cb[l], W)
        if l == 0:
            args = [edges_feat, gsrc, edst_, edges_num_norm_sqrt, cw, cb,
                    wep, bep]
            y_prev, m, st_prev = _edge_main(0, args)
        elif l == 1:
            args = [edges_feat, y_prev, st_prev,
                    _padb(bn_e_gamma[0], W), _padb(bn_e_beta[0], W),
                    gsrc, edst_, edges_num_norm_sqrt, cw, cb, wep, bep]
            e_prev, y_prev, m, st_prev = _edge_main(1, args)
        elif l == 2:
            args = [e_prev, y_prev, st_prev,
                    _padb(bn_e_gamma[1], W), _padb(bn_e_beta[1], W),
                    gsrc, edst_, edges_num_norm_sqrt, cw, cb]
            e_prev, y_prev, m, st_prev = _edge_main(2, args)
        else:
            args = [e_prev, y_prev, st_prev,
                    _padb(bn_e_gamma[2], W), _padb(bn_e_beta[2], W),
                    gsrc, edst_, cw, cb]
            (m,) = _edge_main(3, args)
        p = _sc_scatter(m, dst3, zer)
        h = _node_finish(ah, p, nodes_num_norm_sqrt, h,
                         _padb(bn_h_gamma[l], W), _padb(bn_h_beta[l], W))

    out = _readout(h, _padw(W0, W, 128), _padb(b0, 128),
                   _padw(W1, 128, 128), _padb(b1, 128),
                   _padw(W2, 128, 128), _padb(b2, 128))
    return out[0, :10]


# speedup vs baseline: 4.9603x; 1.0441x over previous
import functools

import jax
import jax.numpy as jnp
from jax import lax
from jax.experimental import pallas as pl
from jax.experimental.pallas import tpu as pltpu
from jax.experimental.pallas import tpu_sc as plsc

N = 10000
E = 320000
W = 128
WD = 256
L = 4
BE = 4000
NBLK = E // BE
NW = 32
CH = 1280
NCHUNK = E // CH
CPW = -(-NCHUNK // NW)
CPS = -(-NCHUNK // 16)
NST = CH // W


def _sc_gather(tab, idx2):
    mesh = plsc.VectorSubcoreMesh(core_axis_name="c", subcore_axis_name="s")

    @functools.partial(
        pl.kernel,
        out_type=jax.ShapeDtypeStruct((2, E, W), jnp.uint32),
        mesh=mesh,
        scratch_types=[
            pltpu.VMEM((NST, 128), jnp.int32),
            pltpu.VMEM((2, 128, W), jnp.uint32),
            pltpu.VMEM_SHARED((N, W), jnp.uint32),
            pltpu.SemaphoreType.DMA,
            pltpu.SemaphoreType.DMA,
        ],
    )
    def k(tab_hbm, idx_hbm, out_hbm, idxv, rows, stab, semg, semw):
        core = lax.axis_index("c")
        s = lax.axis_index("s")

        @pl.when(s < 10)
        def _():
            pltpu.sync_copy(tab_hbm.at[core, pl.ds(s * 1000, 1000)],
                            stab.at[pl.ds(s * 1000, 1000)])

        plsc.subcore_barrier()

        @pl.loop(0, CPS)
        def _(i):
            c = s + i * 16

            @pl.when(c < NCHUNK)
            def _():
                pltpu.sync_copy(idx_hbm.at[core, c], idxv)

                def gath(j):
                    return pltpu.async_copy(
                        stab.at[idxv.at[j]], rows.at[j % 2], semg)

                def wr(j):
                    base = c * CH + j * 128
                    return pltpu.async_copy(
                        rows.at[j % 2], out_hbm.at[core, pl.ds(base, 128)],
                        semw)

                gcur = gath(0)
                wprev = None
                for j in range(NST):
                    gcur.wait()
                    if wprev is not None:
                        wprev.wait()
                    if j + 1 < NST:
                        gnext = gath(j + 1)
                    wcur = wr(j)
                    if j + 1 < NST:
                        gcur = gnext
                    wprev = wcur
                wprev.wait()

    return k(tab, idx2)


def _sc_scatter(m, dst3, zer):
    mesh = plsc.VectorSubcoreMesh(core_axis_name="c", subcore_axis_name="s")

    @functools.partial(
        pl.kernel,
        out_type=jax.ShapeDtypeStruct((2, N, W), jnp.float32),
        mesh=mesh,
        scratch_types=[
            pltpu.VMEM((NST, 128), jnp.int32),
            pltpu.VMEM((2, 128, W), jnp.float32),
            pltpu.VMEM_SHARED((N, W), jnp.float32),
            pltpu.SemaphoreType.DMA,
            pltpu.SemaphoreType.DMA,
        ],
    )
    def k(m_hbm, dst_hbm, z_hbm, p_hbm, didx, mrows, accum, semld, semsc):
        core = lax.axis_index("c")
        s = lax.axis_index("s")

        @pl.when(s == 0)
        def _():
            pltpu.sync_copy(z_hbm, accum)

        plsc.subcore_barrier()

        @pl.loop(0, CPS)
        def _(i):
            c = s + i * 16

            @pl.when(c < NCHUNK)
            def _():
                pltpu.sync_copy(dst_hbm.at[c], didx)

                def mload(j):
                    return pltpu.async_copy(
                        m_hbm.at[core, pl.ds(c * CH + j * 128, 128)],
                        mrows.at[j % 2], semld)

                lcur = mload(0)
                scprev = None
                for j in range(NST):
                    lcur.wait()
                    if scprev is not None:
                        scprev.wait()
                    if j + 1 < NST:
                        lnext = mload(j + 1)
                    sccur = pltpu.async_copy(
                        mrows.at[j % 2], accum.at[didx.at[j]], semsc, add=True)
                    if j + 1 < NST:
                        lcur = lnext
                    scprev = sccur
                scprev.wait()

        plsc.subcore_barrier()

        @pl.when(s < 10)
        def _():
            pltpu.sync_copy(accum.at[pl.ds(s * 1000, 1000)],
                            p_hbm.at[core, pl.ds(s * 1000, 1000)])

    return k(m, dst3, zer)


def _embed_h(nodes_feat, whp, bhp):
    def body(x_ref, w_ref, b_ref, o_ref):
        o_ref[...] = jnp.dot(x_ref[...], w_ref[...],
                             preferred_element_type=jnp.float32) + b_ref[...]
    return pl.pallas_call(
        body, out_shape=jax.ShapeDtypeStruct((N, W), jnp.float32),
    )(nodes_feat, whp, bhp)


def _node_mm(h, wc, bc):
    def body(h_ref, w_ref, b_ref, ah_ref, t_ref):
        hw = jnp.dot(h_ref[...], w_ref[...],
                     preferred_element_type=jnp.float32) + b_ref[...]
        ah_ref[...] = hw[:, 0:W]
        dhb = lax.bitcast_convert_type(
            hw[:, 2 * W:3 * W].astype(jnp.bfloat16).astype(jnp.float32),
            jnp.uint32)
        bhb = lax.bitcast_convert_type(
            hw[:, W:2 * W].astype(jnp.bfloat16).astype(jnp.float32),
            jnp.uint32)
        t_ref[0] = (dhb & jnp.uint32(0xFFFF0000)) | (bhb >> 16)
        t_ref[1] = lax.bitcast_convert_type(hw[:, 3 * W:4 * W], jnp.uint32)
    return pl.pallas_call(
        body,
        out_shape=(jax.ShapeDtypeStruct((N, W), jnp.float32),
                   jax.ShapeDtypeStruct((2, N, W), jnp.uint32)),
    )(h, wc, bc)


def _edge_main(l, args):
    def body(*refs):
        it = iter(refs)
        if l == 0:
            f_ref = next(it)
        elif l == 1:
            f_ref, yp_ref, stp_ref, gp_ref, bp_ref = (next(it) for _ in range(5))
        else:
            ep_ref, yp_ref, stp_ref, gp_ref, bp_ref = (next(it) for _ in range(5))
        g_ref, ed_ref = next(it), next(it)
        if l < 3:
            n_ref = next(it)
        w_ref, b_ref = next(it), next(it)
        if l <= 1:
            we_ref, be_ref = next(it), next(it)
        if l in (1, 2):
            eo_ref = next(it)
        if l < 3:
            y_ref = next(it)
        m_ref = next(it)
        if l < 3:
            st_ref = next(it)

        if l == 0:
            e_in = f_ref[...] * we_ref[...] + be_ref[...]
        else:
            if l == 1:
                e_base = f_ref[...] * we_ref[...] + be_ref[...]
            else:
                e_base = ep_ref[...]
            mu = stp_ref[0:1, :] * (1.0 / E)
            var = stp_ref[1:2, :] * (1.0 / E) - mu * mu
            bn = ((yp_ref[...].astype(jnp.float32) - mu)
                  / jnp.sqrt(var + 1e-5) * gp_ref[...] + bp_ref[...])
            e_in = e_base + jnp.maximum(bn, 0.0)
        if l in (1, 2):
            eo_ref[...] = e_in

        i = pl.program_id(0)
        ce = jnp.dot(e_in, w_ref[...],
                     preferred_element_type=jnp.float32) + b_ref[...]
        gu = g_ref[0]
        d = lax.bitcast_convert_type(gu & jnp.uint32(0xFFFF0000), jnp.float32)
        bmsg = lax.bitcast_convert_type(gu << 16, jnp.float32)
        x = ce + d + lax.bitcast_convert_type(ed_ref[0], jnp.float32)
        sig = jax.nn.sigmoid(x)
        m_ref[0] = sig
        m_ref[1] = sig * bmsg
        if l < 3:
            y = x * n_ref[...]
            y_ref[...] = y.astype(jnp.bfloat16)
            s1 = jnp.sum(y, axis=0, keepdims=True)
            s2 = jnp.sum(y * y, axis=0, keepdims=True)
            part = jnp.concatenate(
                [s1, s2, jnp.zeros((6, W), jnp.float32)], axis=0)

            @pl.when(i == 0)
            def _():
                st_ref[...] = part

            @pl.when(i > 0)
            def _():
                st_ref[...] += part

    eblk = lambda w=W: pl.BlockSpec((BE, w), lambda i: (i, 0))
    cblk = lambda r: pl.BlockSpec((r, W), lambda i: (0, 0))
    in_specs = []
    if l == 0:
        in_specs += [pl.BlockSpec((BE, 1), lambda i: (i, 0))]
    elif l == 1:
        in_specs += [pl.BlockSpec((BE, 1), lambda i: (i, 0)),
                     eblk(), cblk(8), cblk(1), cblk(1)]
    else:
        in_specs += [eblk(), eblk(), cblk(8), cblk(1), cblk(1)]
    in_specs += [pl.BlockSpec((1, BE, W), lambda i: (0, i, 0)),
                 pl.BlockSpec((1, BE, W), lambda i: (1, i, 0))]
    if l < 3:
        in_specs += [pl.BlockSpec((BE, 1), lambda i: (i, 0))]
    in_specs += [cblk(W), cblk(1)]
    if l <= 1:
        in_specs += [cblk(1), cblk(1)]

    out_specs, out_shape = [], []
    if l in (1, 2):
        out_specs += [eblk()]
        out_shape += [jax.ShapeDtypeStruct((E, W), jnp.float32)]
    if l < 3:
        out_specs += [eblk()]
        out_shape += [jax.ShapeDtypeStruct((E, W), jnp.bfloat16)]
    out_specs += [pl.BlockSpec((2, BE, W), lambda i: (0, i, 0))]
    out_shape += [jax.ShapeDtypeStruct((2, E, W), jnp.float32)]
    if l < 3:
        out_specs += [cblk(8)]
        out_shape += [jax.ShapeDtypeStruct((8, W), jnp.float32)]

    return pl.pallas_call(
        body,
        grid=(NBLK,),
        in_specs=in_specs,
        out_specs=tuple(out_specs),
        out_shape=tuple(out_shape),
    )(*args)


def _node_finish(ah, p, nnorm, h_in, gam, bet):
    def body(ah_ref, p_ref, nn_ref, h_ref, g_ref, b_ref, o_ref):
        den = p_ref[0]
        num = p_ref[1]
        hn = (ah_ref[...] + num / (den + 1e-6)) * nn_ref[...]
        mu = jnp.mean(hn, axis=0, keepdims=True)
        var = jnp.mean(hn * hn, axis=0, keepdims=True) - mu * mu
        bn = (hn - mu) / jnp.sqrt(var + 1e-5) * g_ref[...] + b_ref[...]
        o_ref[...] = h_ref[...] + jnp.maximum(bn, 0.0)
    return pl.pallas_call(
        body, out_shape=jax.ShapeDtypeStruct((N, W), jnp.float32),
    )(ah, p, nnorm, h_in, gam, bet)


def _readout(h, w0, b0_, w1, b1_, w2, b2_):
    def body(h_ref, w0_ref, b0_ref, w1_ref, b1_ref, w2_ref, b2_ref, o_ref):
        hg = jnp.mean(h_ref[...], axis=0, keepdims=True)
        y0 = jnp.maximum(jnp.dot(hg, w0_ref[...],
                                 preferred_element_type=jnp.float32) + b0_ref[...], 0.0)
        y1 = jnp.maximum(jnp.dot(y0, w1_ref[...],
                                 preferred_element_type=jnp.float32) + b1_ref[...], 0.0)
        o_ref[...] = jnp.dot(y1, w2_ref[...],
                             preferred_element_type=jnp.float32) + b2_ref[...]
    return pl.pallas_call(
        body, out_shape=jax.ShapeDtypeStruct((1, 128), jnp.float32),
    )(h, w0, b0_, w1, b1_, w2, b2_)


def _padw(w, r, c):
    return jnp.zeros((r, c), jnp.float32).at[:w.shape[0], :w.shape[1]].set(w)


def _padb(b, c):
    return jnp.zeros((1, c), jnp.float32).at[0, :b.shape[0]].set(b)


def kernel(nodes_feat, edges_feat, nodes_num_norm_sqrt, edges_num_norm_sqrt, edge_index, Wh, bh, We, be, AW, Ab, BW, Bb, CW, Cb, DW, Db, EW, Eb, bn_h_gamma, bn_h_beta, bn_e_gamma, bn_e_beta, W0, b0, W1, b1, W2, b2):
    idx2 = edge_index.reshape(2, NCHUNK, NST, 128)
    dst3 = edge_index[1].reshape(NCHUNK, NST, 128)

    h = _embed_h(nodes_feat, _padw(Wh, 128, W), _padb(bh, W))
    zer = jnp.zeros((N, W), jnp.float32)
    wep = _padw(We, 1, W)
    bep = _padb(be, W)

    y_prev = st_prev = e_prev = None
    for l in range(L):
        wc = jnp.concatenate([_padw(AW[l], W, W), _padw(BW[l], W, W),
                              _padw(DW[l], W, W), _padw(EW[l], W, W)], axis=1)
        bc = jnp.concatenate([_padb(Ab[l], W), _padb(Bb[l], W),
                              _padb(Db[l], W), _padb(Eb[l], W)], axis=1)
        ah, tab = _node_mm(h, wc, bc)
        gout = _sc_gather(tab, idx2)
        gsrc = edst_ = gout
        cw = _padw(CW[l], W, W)
        cb = _padb(Cb[l], W)
        if l == 0:
            args = [edges_feat, gsrc, edst_, edges_num_norm_sqrt, cw, cb,
                    wep, bep]
            y_prev, m, st_prev = _edge_main(0, args)
        elif l == 1:
            args = [edges_feat, y_prev, st_prev,
                    _padb(bn_e_gamma[0], W), _padb(bn_e_beta[0], W),
                    gsrc, edst_, edges_num_norm_sqrt, cw, cb, wep, bep]
            e_prev, y_prev, m, st_prev = _edge_main(1, args)
        elif l == 2:
            args = [e_prev, y_prev, st_prev,
                    _padb(bn_e_gamma[1], W), _padb(bn_e_beta[1], W),
                    gsrc, edst_, edges_num_norm_sqrt, cw, cb]
            e_prev, y_prev, m, st_prev = _edge_main(2, args)
        else:
            args = [e_prev, y_prev, st_prev,
                    _padb(bn_e_gamma[2], W), _padb(bn_e_beta[2], W),
                    gsrc, edst_, cw, cb]
            (m,) = _edge_main(3, args)
        p = _sc_scatter(m, dst3, zer)
        h = _node_finish(ah, p, nodes_num_norm_sqrt, h,
                         _padb(bn_h_gamma[l], W), _padb(bn_h_beta[l], W))

    out = _readout(h, _padw(W0, W, 128), _padb(b0, 128),
                   _padw(W1, 128, 128), _padb(b1, 128),
                   _padw(W2, 128, 128), _padb(b2, 128))
    return out[0, :10]
